# baseline (speedup 1.0000x reference)
import functools

import jax
import jax.numpy as jnp
from jax import lax
from jax.experimental import pallas as pl
from jax.experimental.pallas import tpu as pltpu
from jax.experimental.pallas import tpu_sc as plsc

B = 4096
S = 32
H = 128
L = 16

NC = 2
NS = 16
NW = NC * NS
RPW = B // NW
RPC = 4
G = RPC * S
CW = RPW // RPC
TPW = RPW * S
HC = H // L


@functools.partial(
    pl.kernel,
    out_type=jax.ShapeDtypeStruct((B, H), jnp.float32),
    mesh=plsc.VectorSubcoreMesh(core_axis_name="c", subcore_axis_name="s"),
    compiler_params=pltpu.CompilerParams(needs_layout_passes=False),
    scratch_types=[
        pltpu.VMEM((CW, G), jnp.int32),
        pltpu.VMEM((TPW,), jnp.int32),
        pltpu.VMEM((2, G, H), jnp.float32),
        pltpu.VMEM((1, H), jnp.float32),
        pltpu.VMEM((RPW, H), jnp.float32),
        pltpu.VMEM((RPW,), jnp.float32),
        pltpu.SemaphoreType.DMA,
        pltpu.SemaphoreType.DMA,
    ],
)
def _pool_kernel(table_hbm, ids_hbm, mask_hbm, out_hbm,
                 ids_v, mask_v, rows_v, t0_v, out_v, cnt_v, sem0, sem1):
    w = lax.axis_index("s") * NC + lax.axis_index("c")

    pltpu.sync_copy(ids_hbm.at[w], ids_v)
    pltpu.sync_copy(mask_hbm.at[w], mask_v)
    pltpu.sync_copy(table_hbm.at[pl.ds(0, 1)], t0_v)

    def _premul(i, carry):
        c = i // (G // L)
        j = (i % (G // L)) * L
        mv = mask_v[pl.ds(i * L, L)]
        ids_v[c, pl.ds(j, L)] = ids_v[c, pl.ds(j, L)] * mv
        return carry

    lax.fori_loop(0, TPW // L, _premul, 0)

    lanes = lax.iota(jnp.int32, L)

    def _count(k, carry):
        def _cs(s, acc):
            idx = lanes * S + (k * (L * S) + s)
            return acc + plsc.load_gather(mask_v, [idx])

        acc = lax.fori_loop(0, S, _cs, jnp.zeros((L,), jnp.int32))
        cnt_v[pl.ds(k * L, L)] = acc.astype(jnp.float32)
        return carry

    lax.fori_loop(0, RPW // L, _count, 0)

    sems = (sem0, sem1)

    def start(c):
        return pltpu.async_copy(
            table_hbm.at[ids_v.at[c]], rows_v.at[c % 2], sems[c % 2])

    def process(c):
        b = c % 2

        def row_body(r, carry):
            base = r * S

            def s_body(s, acc):
                row = base + s
                return tuple(
                    acc[h] + rows_v[b, row, pl.ds(h * L, L)]
                    for h in range(HC))

            acc = lax.fori_loop(
                0, S, s_body,
                tuple(jnp.zeros((L,), jnp.float32) for _ in range(HC)))

            g = c * RPC + r
            cntf = plsc.load_gather(cnt_v, [jnp.full((L,), g, jnp.int32)])
            n0f = jnp.float32(S) - cntf
            inv = 1.0 / jnp.maximum(cntf, 1.0)
            for h in range(HC):
                out_v[g, pl.ds(h * L, L)] = (
                    acc[h] - n0f * t0_v[0, pl.ds(h * L, L)]) * inv
            return carry

        lax.fori_loop(0, RPC, row_body, 0)

    cps = [start(0), start(1)]
    for c in range(CW):
        cps[c % 2].wait()
        process(c)
        if c + 2 < CW:
            cps[c % 2] = start(c + 2)

    pltpu.sync_copy(out_v, out_hbm.at[pl.ds(w * RPW, RPW)])


def kernel(ids, mask, embed_table):
    ids_r = ids.reshape(NW, CW, G)
    mask_r = mask.reshape(NW, TPW)
    return _pool_kernel(embed_table, ids_r, mask_r)


# speedup vs baseline: 37.9830x; 37.9830x over previous
import functools

import jax
import jax.numpy as jnp
from jax import lax
from jax.experimental import pallas as pl
from jax.experimental.pallas import tpu as pltpu
from jax.experimental.pallas import tpu_sc as plsc

B = 4096
S = 32
H = 128
L = 16

NC = 2
NS = 16
NW = NC * NS
RPW = B // NW
RPC = 4
G = RPC * S
CW = RPW // RPC
TPW = RPW * S
HC = H // L


@functools.partial(
    pl.kernel,
    out_type=jax.ShapeDtypeStruct((B, H), jnp.float32),
    mesh=plsc.VectorSubcoreMesh(core_axis_name="c", subcore_axis_name="s"),
    compiler_params=pltpu.CompilerParams(needs_layout_passes=False),
    scratch_types=[
        pltpu.VMEM((CW, G), jnp.int32),
        pltpu.VMEM((TPW,), jnp.int32),
        pltpu.VMEM((2, G, H), jnp.float32),
        pltpu.VMEM((RPW, H), jnp.float32),
        pltpu.VMEM((RPW,), jnp.float32),
        pltpu.VMEM((TPW,), jnp.float32),
        pltpu.SemaphoreType.DMA,
        pltpu.SemaphoreType.DMA,
    ],
)
def _pool_kernel(table_hbm, ids_hbm, mask_hbm, out_hbm,
                 ids_v, mask_v, rows_v, out_v, inv_v, w_v, sem0, sem1):
    w = lax.axis_index("s") * NC + lax.axis_index("c")

    pltpu.sync_copy(ids_hbm.at[w], ids_v)
    pltpu.sync_copy(mask_hbm.at[w], mask_v)

    lanes = lax.iota(jnp.int32, L)

    def _count(k, carry):
        def _cs(s, acc):
            idx = lanes * S + (k * (L * S) + s)
            return acc + plsc.load_gather(mask_v, [idx])

        acc = lax.fori_loop(0, S, _cs, jnp.zeros((L,), jnp.int32))
        inv_v[pl.ds(k * L, L)] = 1.0 / jnp.maximum(
            acc.astype(jnp.float32), 1.0)
        return carry

    lax.fori_loop(0, RPW // L, _count, 0)

    def _weight(i, carry):
        ginv = plsc.load_gather(inv_v, [jnp.full((L,), i // 2, jnp.int32)])
        mv = mask_v[pl.ds(i * L, L)].astype(jnp.float32)
        w_v[pl.ds(i * L, L)] = mv * ginv
        return carry

    lax.fori_loop(0, TPW // L, _weight, 0)

    sems = (sem0, sem1)

    def start(c):
        return pltpu.async_copy(
            table_hbm.at[ids_v.at[c]], rows_v.at[c % 2], sems[c % 2])

    def process(c):
        b = c % 2

        def row_body(r, carry):
            base = r * S
            tok0 = (c * RPC + r) * S

            def s_body(s, acc):
                row = base + s
                wv = plsc.load_gather(w_v, [jnp.full((L,), tok0 + s, jnp.int32)])
                return tuple(
                    acc[h] + wv * rows_v[b, row, pl.ds(h * L, L)]
                    for h in range(HC))

            acc = lax.fori_loop(
                0, S, s_body,
                tuple(jnp.zeros((L,), jnp.float32) for _ in range(HC)))

            g = c * RPC + r
            for h in range(HC):
                out_v[g, pl.ds(h * L, L)] = acc[h]
            return carry

        lax.fori_loop(0, RPC, row_body, 0)

    cps = [start(0), start(1)]
    for c in range(CW):
        cps[c % 2].wait()
        process(c)
        if c + 2 < CW:
            cps[c % 2] = start(c + 2)

    pltpu.sync_copy(out_v, out_hbm.at[pl.ds(w * RPW, RPW)])


def kernel(ids, mask, embed_table):
    ids_r = ids.reshape(NW, CW, G)
    mask_r = mask.reshape(NW, TPW)
    return _pool_kernel(embed_table, ids_r, mask_r)


# speedup vs baseline: 43.7095x; 1.1508x over previous
import functools

import jax
import jax.numpy as jnp
from jax import lax
from jax.experimental import pallas as pl
from jax.experimental.pallas import tpu as pltpu
from jax.experimental.pallas import tpu_sc as plsc

B = 4096
S = 32
H = 128
L = 16

NC = 2
NS = 16
NW = NC * NS
RPW = B // NW
RPC = 4
G = RPC * S
CW = RPW // RPC
TPW = RPW * S
HC = H // L


@functools.partial(
    pl.kernel,
    out_type=jax.ShapeDtypeStruct((B, H), jnp.float32),
    mesh=plsc.VectorSubcoreMesh(core_axis_name="c", subcore_axis_name="s"),
    compiler_params=pltpu.CompilerParams(needs_layout_passes=False),
    scratch_types=[
        pltpu.VMEM((CW, G), jnp.int32),
        pltpu.VMEM((TPW,), jnp.int32),
        pltpu.VMEM((4, G, H), jnp.float32),
        pltpu.VMEM((RPW, H), jnp.float32),
        pltpu.VMEM((RPW,), jnp.float32),
        pltpu.VMEM((TPW,), jnp.float32),
        pltpu.SemaphoreType.DMA,
        pltpu.SemaphoreType.DMA,
        pltpu.SemaphoreType.DMA,
        pltpu.SemaphoreType.DMA,
    ],
)
def _pool_kernel(table_hbm, ids_hbm, mask_hbm, out_hbm,
                 ids_v, mask_v, rows_v, out_v, inv_v, w_v,
                 sem0, sem1, sem2, sem3):
    w = lax.axis_index("s") * NC + lax.axis_index("c")

    pltpu.sync_copy(ids_hbm.at[w], ids_v)
    pltpu.sync_copy(mask_hbm.at[w], mask_v)

    lanes = lax.iota(jnp.int32, L)

    def _count(k, carry):
        def _cs(s, acc):
            idx = lanes * S + (k * (L * S) + s)
            return acc + plsc.load_gather(mask_v, [idx])

        acc = lax.fori_loop(0, S, _cs, jnp.zeros((L,), jnp.int32))
        inv_v[pl.ds(k * L, L)] = 1.0 / jnp.maximum(
            acc.astype(jnp.float32), 1.0)
        return carry

    lax.fori_loop(0, RPW // L, _count, 0)

    def _weight(i, carry):
        ginv = plsc.load_gather(inv_v, [jnp.full((L,), i // 2, jnp.int32)])
        mv = mask_v[pl.ds(i * L, L)].astype(jnp.float32)
        w_v[pl.ds(i * L, L)] = mv * ginv
        return carry

    lax.fori_loop(0, TPW // L, _weight, 0)

    sems = (sem0, sem1, sem2, sem3)
    NBUF = 4
    AHEAD = 3

    def start(c):
        return pltpu.async_copy(
            table_hbm.at[ids_v.at[c]], rows_v.at[c % NBUF], sems[c % NBUF])

    def process(c):
        b = c % NBUF

        def row_body(r, carry):
            base = r * S
            tok0 = (c * RPC + r) * S

            def s_body(s, acc):
                row = base + s
                wv = plsc.load_gather(w_v, [jnp.full((L,), tok0 + s, jnp.int32)])
                return tuple(
                    acc[h] + wv * rows_v[b, row, pl.ds(h * L, L)]
                    for h in range(HC))

            acc = lax.fori_loop(
                0, S, s_body,
                tuple(jnp.zeros((L,), jnp.float32) for _ in range(HC)))

            g = c * RPC + r
            for h in range(HC):
                out_v[g, pl.ds(h * L, L)] = acc[h]
            return carry

        lax.fori_loop(0, RPC, row_body, 0)

    cps = {c: start(c) for c in range(AHEAD)}
    for c in range(CW):
        cps[c].wait()
        process(c)
        if c + AHEAD < CW:
            cps[c + AHEAD] = start(c + AHEAD)

    pltpu.sync_copy(out_v, out_hbm.at[pl.ds(w * RPW, RPW)])


def kernel(ids, mask, embed_table):
    ids_r = ids.reshape(NW, CW, G)
    mask_r = mask.reshape(NW, TPW)
    return _pool_kernel(embed_table, ids_r, mask_r)


# speedup vs baseline: 44.1883x; 1.0110x over previous
import functools

import jax
import jax.numpy as jnp
from jax import lax
from jax.experimental import pallas as pl
from jax.experimental.pallas import tpu as pltpu
from jax.experimental.pallas import tpu_sc as plsc

B = 4096
S = 32
H = 128
L = 16

NC = 2
NS = 16
NW = NC * NS
RPW = B // NW
RPC = 4
G = RPC * S
CW = RPW // RPC
TPW = RPW * S
HC = H // L


@functools.partial(
    pl.kernel,
    out_type=jax.ShapeDtypeStruct((B, H), jnp.float32),
    mesh=plsc.VectorSubcoreMesh(core_axis_name="c", subcore_axis_name="s"),
    compiler_params=pltpu.CompilerParams(needs_layout_passes=False),
    scratch_types=[
        pltpu.VMEM((CW, G), jnp.int32),
        pltpu.VMEM((TPW,), jnp.int32),
        pltpu.VMEM((6, G, H), jnp.float32),
        pltpu.VMEM((RPW, H), jnp.float32),
        pltpu.VMEM((RPW,), jnp.float32),
        pltpu.VMEM((TPW,), jnp.float32),
        pltpu.SemaphoreType.DMA,
        pltpu.SemaphoreType.DMA,
        pltpu.SemaphoreType.DMA,
        pltpu.SemaphoreType.DMA,
        pltpu.SemaphoreType.DMA,
        pltpu.SemaphoreType.DMA,
    ],
)
def _pool_kernel(table_hbm, ids_hbm, mask_hbm, out_hbm,
                 ids_v, mask_v, rows_v, out_v, inv_v, w_v,
                 sem0, sem1, sem2, sem3, sem4, sem5):
    w = lax.axis_index("s") * NC + lax.axis_index("c")

    pltpu.sync_copy(ids_hbm.at[w], ids_v)
    pltpu.sync_copy(mask_hbm.at[w], mask_v)

    lanes = lax.iota(jnp.int32, L)

    def _count(k, carry):
        def _cs(s, acc):
            idx = lanes * S + (k * (L * S) + s)
            return acc + plsc.load_gather(mask_v, [idx])

        acc = lax.fori_loop(0, S, _cs, jnp.zeros((L,), jnp.int32))
        inv_v[pl.ds(k * L, L)] = 1.0 / jnp.maximum(
            acc.astype(jnp.float32), 1.0)
        return carry

    lax.fori_loop(0, RPW // L, _count, 0)

    def _weight(i, carry):
        ginv = plsc.load_gather(inv_v, [jnp.full((L,), i // 2, jnp.int32)])
        mv = mask_v[pl.ds(i * L, L)].astype(jnp.float32)
        w_v[pl.ds(i * L, L)] = mv * ginv
        return carry

    lax.fori_loop(0, TPW // L, _weight, 0)

    sems = (sem0, sem1, sem2, sem3, sem4, sem5)
    NBUF = 6
    AHEAD = 5

    def start(c):
        return pltpu.async_copy(
            table_hbm.at[ids_v.at[c]], rows_v.at[c % NBUF], sems[c % NBUF])

    def process(c):
        b = c % NBUF

        def row_body(r, carry):
            base = r * S
            tok0 = (c * RPC + r) * S

            def s_body(s, acc):
                row = base + s
                wv = plsc.load_gather(w_v, [jnp.full((L,), tok0 + s, jnp.int32)])
                return tuple(
                    acc[h] + wv * rows_v[b, row, pl.ds(h * L, L)]
                    for h in range(HC))

            acc = lax.fori_loop(
                0, S, s_body,
                tuple(jnp.zeros((L,), jnp.float32) for _ in range(HC)))

            g = c * RPC + r
            for h in range(HC):
                out_v[g, pl.ds(h * L, L)] = acc[h]
            return carry

        lax.fori_loop(0, RPC, row_body, 0)

    cps = {c: start(c) for c in range(AHEAD)}
    for c in range(CW):
        cps[c].wait()
        process(c)
        if c + AHEAD < CW:
            cps[c + AHEAD] = start(c + AHEAD)

    pltpu.sync_copy(out_v, out_hbm.at[pl.ds(w * RPW, RPW)])


def kernel(ids, mask, embed_table):
    ids_r = ids.reshape(NW, CW, G)
    mask_r = mask.reshape(NW, TPW)
    return _pool_kernel(embed_table, ids_r, mask_r)


# speedup vs baseline: 46.4573x; 1.0514x over previous
import functools

import jax
import jax.numpy as jnp
from jax import lax
from jax.experimental import pallas as pl
from jax.experimental.pallas import tpu as pltpu
from jax.experimental.pallas import tpu_sc as plsc

B = 4096
S = 32
H = 128
L = 16

NC = 2
NS = 16
NW = NC * NS
RPW = B // NW
G = 128
CW = RPW * S // G
TPW = RPW * S
HC = H // L
NBUF = 5
AHEAD = 4
RING = NBUF * G


@functools.partial(
    pl.kernel,
    out_type=jax.ShapeDtypeStruct((B, H), jnp.float32),
    mesh=plsc.VectorSubcoreMesh(core_axis_name="c", subcore_axis_name="s"),
    compiler_params=pltpu.CompilerParams(
        needs_layout_passes=False, use_tc_tiling_on_sc=False),
    scratch_types=[
        pltpu.VMEM((TPW,), jnp.int32),
        pltpu.VMEM((TPW,), jnp.int32),
        pltpu.VMEM((RING, H), jnp.float32),
        pltpu.VMEM((RPW, H), jnp.float32),
        pltpu.VMEM((RPW + 1, L), jnp.int32),
        pltpu.VMEM((RPW, L), jnp.float32),
        pltpu.SemaphoreType.DMA,
        pltpu.SemaphoreType.DMA,
        pltpu.SemaphoreType.DMA,
        pltpu.SemaphoreType.DMA,
        pltpu.SemaphoreType.DMA,
    ],
)
def _pool_kernel(table_hbm, ids_hbm, mask_hbm, out_hbm,
                 packed_v, mask_v, rows_v, out_v, offx_v, invx_v,
                 sem0, sem1, sem2, sem3, sem4):
    w = lax.axis_index("s") * NC + lax.axis_index("c")

    pltpu.sync_copy(ids_hbm.at[w], packed_v)
    pltpu.sync_copy(mask_hbm.at[w], mask_v)

    def pack_row(g, off):
        offx_v[g] = jnp.full((L,), 0, jnp.int32) + off

        def chunk(j, off_):
            i = g * 2 + j
            mv = mask_v[pl.ds(i * L, L)]
            keep = mv != 0
            incl = plsc.cumsum(mv)
            pos = off_ + incl - 1
            ids_c = packed_v[pl.ds(i * L, L)]
            plsc.store_scatter(packed_v, [pos], ids_c, mask=keep)
            return off_ + incl[L - 1]

        off2 = lax.fori_loop(0, 2, chunk, off)
        cntv = jnp.full((L,), 0.0, jnp.float32) + (off2 - off).astype(jnp.float32)
        invx_v[g] = 1.0 / jnp.maximum(cntv, 1.0)
        return off2

    kept = lax.fori_loop(0, RPW, pack_row, jnp.int32(0))
    offx_v[RPW] = jnp.full((L,), 0, jnp.int32) + kept

    sems = (sem0, sem1, sem2, sem3, sem4)

    def dcopy(c):
        return pltpu.make_async_copy(
            table_hbm.at[packed_v.at[pl.ds(c * G, G)]],
            rows_v.at[pl.ds((c % NBUF) * G, G)],
            sems[c % NBUF])

    def row_loop_body(carry):
        g, st, en = carry

        def s_body(p, acc):
            pidx = acc[HC]
            new = tuple(
                acc[h] + rows_v[pidx, pl.ds(h * L, L)] for h in range(HC))
            pidx1 = pidx + 1
            pidx1 = jnp.where(pidx1 == RING, 0, pidx1)
            return (*new, pidx1)

        pidx0 = st - (st // RING) * RING
        res = lax.fori_loop(
            st, en, s_body,
            tuple(jnp.zeros((L,), jnp.float32) for _ in range(HC)) + (pidx0,))
        invv = invx_v[g]
        for h in range(HC):
            out_v[g, pl.ds(h * L, L)] = res[h] * invv
        en_next = offx_v[jnp.minimum(g + 2, RPW)][0]
        return (g + 1, en, en_next)

    def process_rows(carry, limit):
        def cond(carry):
            g, st, en = carry
            return (g < RPW) & (en <= limit)

        return lax.while_loop(cond, row_loop_body, carry)

    for c in range(min(AHEAD, CW)):
        @pl.when(c * G < kept)
        def _(c=c):
            dcopy(c).start()

    carry = (jnp.int32(0), jnp.int32(0), offx_v[jnp.minimum(1, RPW)][0])
    for c in range(CW):
        @pl.when(c * G < kept)
        def _(c=c):
            dcopy(c).wait()
        carry = process_rows(carry, (c + 1) * G)
        if c + AHEAD < CW:
            @pl.when((c + AHEAD) * G < kept)
            def _(c=c):
                dcopy(c + AHEAD).start()

    pltpu.sync_copy(out_v, out_hbm.at[pl.ds(w * RPW, RPW)])


def kernel(ids, mask, embed_table):
    ids_r = ids.reshape(NW, TPW)
    mask_r = mask.reshape(NW, TPW)
    return _pool_kernel(embed_table, ids_r, mask_r)


# speedup vs baseline: 46.5607x; 1.0022x over previous
import functools

import jax
import jax.numpy as jnp
from jax import lax
from jax.experimental import pallas as pl
from jax.experimental.pallas import tpu as pltpu
from jax.experimental.pallas import tpu_sc as plsc

B = 4096
S = 32
H = 128
L = 16

NC = 2
NS = 16
NW = NC * NS
RPW = B // NW
G = 128
CW = RPW * S // G
TPW = RPW * S
HC = H // L
NBUF = 5
AHEAD = 4
RING = NBUF * G


@functools.partial(
    pl.kernel,
    out_type=jax.ShapeDtypeStruct((B, H), jnp.float32),
    mesh=plsc.VectorSubcoreMesh(core_axis_name="c", subcore_axis_name="s"),
    compiler_params=pltpu.CompilerParams(
        needs_layout_passes=False, use_tc_tiling_on_sc=False,
        skip_device_barrier=True),
    scratch_types=[
        pltpu.VMEM((TPW,), jnp.int32),
        pltpu.VMEM((TPW,), jnp.int32),
        pltpu.VMEM((RING, H), jnp.float32),
        pltpu.VMEM((RPW, H), jnp.float32),
        pltpu.VMEM((RPW + 1, L), jnp.int32),
        pltpu.VMEM((RPW, L), jnp.float32),
        pltpu.SemaphoreType.DMA,
        pltpu.SemaphoreType.DMA,
        pltpu.SemaphoreType.DMA,
        pltpu.SemaphoreType.DMA,
        pltpu.SemaphoreType.DMA,
    ],
)
def _pool_kernel(table_hbm, ids_hbm, mask_hbm, out_hbm,
                 packed_v, mask_v, rows_v, out_v, offx_v, invx_v,
                 sem0, sem1, sem2, sem3, sem4):
    w = lax.axis_index("s") * NC + lax.axis_index("c")

    pltpu.sync_copy(ids_hbm.at[w], packed_v)
    pltpu.sync_copy(mask_hbm.at[w], mask_v)

    def pack_row(g, off):
        offx_v[g] = jnp.full((L,), 0, jnp.int32) + off

        def chunk(j, off_):
            i = g * 2 + j
            mv = mask_v[pl.ds(i * L, L)]
            keep = mv != 0
            incl = plsc.cumsum(mv)
            pos = off_ + incl - 1
            ids_c = packed_v[pl.ds(i * L, L)]
            plsc.store_scatter(packed_v, [pos], ids_c, mask=keep)
            return off_ + incl[L - 1]

        off2 = lax.fori_loop(0, 2, chunk, off)
        cntv = jnp.full((L,), 0.0, jnp.float32) + (off2 - off).astype(jnp.float32)
        invx_v[g] = 1.0 / jnp.maximum(cntv, 1.0)
        return off2

    kept = lax.fori_loop(0, RPW, pack_row, jnp.int32(0))
    offx_v[RPW] = jnp.full((L,), 0, jnp.int32) + kept

    sems = (sem0, sem1, sem2, sem3, sem4)

    def dcopy(c):
        return pltpu.make_async_copy(
            table_hbm.at[packed_v.at[pl.ds(c * G, G)]],
            rows_v.at[pl.ds((c % NBUF) * G, G)],
            sems[c % NBUF])

    def row_loop_body(carry):
        g, st, en = carry

        def s_body(p, acc):
            pidx = acc[HC]
            new = tuple(
                acc[h] + rows_v[pidx, pl.ds(h * L, L)] for h in range(HC))
            pidx1 = pidx + 1
            pidx1 = jnp.where(pidx1 == RING, 0, pidx1)
            return (*new, pidx1)

        pidx0 = st - (st // RING) * RING
        res = lax.fori_loop(
            st, en, s_body,
            tuple(jnp.zeros((L,), jnp.float32) for _ in range(HC)) + (pidx0,))
        invv = invx_v[g]
        for h in range(HC):
            out_v[g, pl.ds(h * L, L)] = res[h] * invv
        en_next = offx_v[jnp.minimum(g + 2, RPW)][0]
        return (g + 1, en, en_next)

    def process_rows(carry, limit):
        def cond(carry):
            g, st, en = carry
            return (g < RPW) & (en <= limit)

        return lax.while_loop(cond, row_loop_body, carry)

    for c in range(min(AHEAD, CW)):
        @pl.when(c * G < kept)
        def _(c=c):
            dcopy(c).start()

    carry = (jnp.int32(0), jnp.int32(0), offx_v[jnp.minimum(1, RPW)][0])
    for c in range(CW):
        @pl.when(c * G < kept)
        def _(c=c):
            dcopy(c).wait()
        carry = process_rows(carry, (c + 1) * G)
        if c + AHEAD < CW:
            @pl.when((c + AHEAD) * G < kept)
            def _(c=c):
                dcopy(c + AHEAD).start()

    pltpu.sync_copy(out_v, out_hbm.at[pl.ds(w * RPW, RPW)])


def kernel(ids, mask, embed_table):
    ids_r = ids.reshape(NW, TPW)
    mask_r = mask.reshape(NW, TPW)
    return _pool_kernel(embed_table, ids_r, mask_r)


# speedup vs baseline: 53.1978x; 1.1425x over previous
import functools

import jax
import jax.numpy as jnp
from jax import lax
from jax.experimental import pallas as pl
from jax.experimental.pallas import tpu as pltpu
from jax.experimental.pallas import tpu_sc as plsc

B = 4096
S = 32
H = 128
L = 16

NC = 2
NS = 16
NW = NC * NS
RPW = B // NW
G = 128
CW = RPW * S // G
TPW = RPW * S
HC = H // L
NBUF = 5
AHEAD = 4
RING = NBUF * G


@functools.partial(
    pl.kernel,
    out_type=jax.ShapeDtypeStruct((B, H), jnp.float32),
    mesh=plsc.VectorSubcoreMesh(core_axis_name="c", subcore_axis_name="s"),
    compiler_params=pltpu.CompilerParams(
        needs_layout_passes=False, use_tc_tiling_on_sc=False),
    scratch_types=[
        pltpu.VMEM((RPW, S), jnp.int32),
        pltpu.VMEM((TPW,), jnp.int32),
        pltpu.VMEM((RPW, S), jnp.int32),
        pltpu.VMEM((RING, H), jnp.float32),
        pltpu.VMEM((RPW, H), jnp.float32),
        pltpu.VMEM((RPW + 1, L), jnp.int32),
        pltpu.VMEM((RPW, L), jnp.float32),
        pltpu.SemaphoreType.DMA((NBUF,)),
    ],
)
def _pool_kernel(table_hbm, ids_hbm, mask_hbm, out_hbm,
                 ids_v, packed_v, mask_v, rows_v, out_v, offx_v, invx_v,
                 sems):
    w = lax.axis_index("s") * NC + lax.axis_index("c")

    pltpu.sync_copy(ids_hbm.at[pl.ds(w * RPW, RPW)], ids_v)
    pltpu.sync_copy(mask_hbm.at[pl.ds(w * RPW, RPW)], mask_v)

    def pack_row(g, carry):
        off, doff = carry
        offx_v[g] = jnp.full((L,), 0, jnp.int32) + off

        def chunk(j, c_):
            off_, doff_ = c_
            mv = mask_v[g, pl.ds(j * L, L)]
            keep = mv != 0
            incl = plsc.cumsum(mv)
            dincl = plsc.cumsum(1 - mv)
            ids_c = ids_v[g, pl.ds(j * L, L)]
            plsc.store_scatter(packed_v, [off_ + incl - 1], ids_c, mask=keep)
            plsc.store_scatter(
                packed_v, [(TPW - 1 - doff_) - (dincl - 1)], ids_c,
                mask=jnp.logical_not(keep))
            return (off_ + incl[L - 1], doff_ + dincl[L - 1])

        off2, doff2 = lax.fori_loop(0, 2, chunk, (off, doff))
        cntv = jnp.full((L,), 0.0, jnp.float32) + (off2 - off).astype(jnp.float32)
        invx_v[g] = 1.0 / jnp.maximum(cntv, 1.0)
        return (off2, doff2)

    kept, _ = lax.fori_loop(0, RPW, pack_row, (jnp.int32(0), jnp.int32(0)))
    offx_v[RPW] = jnp.full((L,), 0, jnp.int32) + kept
    nch = (kept + (G - 1)) // G

    def dcopy(c):
        boff = (c - (c // NBUF) * NBUF) * G
        return pltpu.make_async_copy(
            table_hbm.at[packed_v.at[pl.ds(c * G, G)]],
            rows_v.at[pl.ds(boff, G)],
            sems.at[c - (c // NBUF) * NBUF])

    def row_loop_body(carry):
        g, st, en = carry

        def s_body(p, acc):
            pidx = acc[HC]
            new = tuple(
                acc[h] + rows_v[pidx, pl.ds(h * L, L)] for h in range(HC))
            pidx1 = pidx + 1
            pidx1 = jnp.where(pidx1 == RING, 0, pidx1)
            return (*new, pidx1)

        pidx0 = st - (st // RING) * RING
        res = lax.fori_loop(
            st, en, s_body,
            tuple(jnp.zeros((L,), jnp.float32) for _ in range(HC)) + (pidx0,))
        invv = invx_v[g]
        for h in range(HC):
            out_v[g, pl.ds(h * L, L)] = res[h] * invv
        en_next = offx_v[jnp.minimum(g + 2, RPW)][0]
        return (g + 1, en, en_next)

    def process_rows(carry, limit):
        def cond(carry):
            g, st, en = carry
            return (g < RPW) & (en <= limit)

        return lax.while_loop(cond, row_loop_body, carry)

    def prime(c, carry):
        @pl.when(c < nch)
        def _():
            dcopy(c).start()
        return carry

    lax.fori_loop(0, AHEAD, prime, 0)

    def step(c, carry):
        @pl.when(c < nch)
        def _():
            dcopy(c).wait()
        carry = process_rows(carry, (c + 1) * G)

        @pl.when(c + AHEAD < nch)
        def _():
            dcopy(c + AHEAD).start()
        return carry

    carry = (jnp.int32(0), jnp.int32(0), offx_v[jnp.minimum(1, RPW)][0])
    lax.fori_loop(0, jnp.maximum(nch, 1), step, carry)

    pltpu.sync_copy(out_v, out_hbm.at[pl.ds(w * RPW, RPW)])


def kernel(ids, mask, embed_table):
    return _pool_kernel(embed_table, ids, mask)
